# baseline (speedup 1.0000x reference)
import functools

import jax
import jax.numpy as jnp
from jax import lax
from jax.experimental import pallas as pl
from jax.experimental.pallas import tpu as pltpu
from jax.experimental.pallas import tpu_sc as plsc

BANK_SIZE = 500000
DIM = 256
NEG_SIZE = 2048
BATCH = 256
TEMP = 0.07

K = NEG_SIZE + 1
CK = 96
NCHUNK = 22
K_PAD = CK * NCHUNK
NC = 2
NS = 16
NW = NC * NS
BPW = BATCH // NW
NJ = DIM // 16


def _tree_sum(vals):
    vals = list(vals)
    while len(vals) > 1:
        nxt = [vals[i] + vals[i + 1] for i in range(0, len(vals) - 1, 2)]
        if len(vals) % 2:
            nxt.append(vals[-1])
        vals = nxt
    return vals[0]


def _sc_body(x_hbm, idx_hbm, mem_hbm, out_hbm,
             xv, idxv, rowsb, part, stage, sem0, sem1):
    cid = lax.axis_index("c")
    sid = lax.axis_index("s")
    wid = sid * NC + cid
    b0 = wid * BPW
    lane = lax.iota(jnp.int32, 16)

    pltpu.sync_copy(x_hbm.at[pl.ds(b0, BPW)], xv)

    def per_b(lb, _):
        row = b0 + lb
        pltpu.sync_copy(idx_hbm.at[row], idxv)
        xs = [xv[lb, pl.ds(16 * j, 16)] for j in range(NJ)]

        pltpu.async_copy(mem_hbm.at[idxv.at[0]], rowsb.at[pl.ds(0, CK)],
                         sem0)

        def chunk_body(c, _):
            par = lax.rem(c, 2)
            even = par == 0

            @pl.when((c < NCHUNK - 1) & even)
            def _():
                pltpu.async_copy(mem_hbm.at[idxv.at[c + 1]],
                                 rowsb.at[pl.ds(CK, CK)], sem1)

            @pl.when((c < NCHUNK - 1) & jnp.logical_not(even))
            def _():
                pltpu.async_copy(mem_hbm.at[idxv.at[c + 1]],
                                 rowsb.at[pl.ds(0, CK)], sem0)

            @pl.when(even)
            def _():
                pltpu.make_async_copy(mem_hbm.at[idxv.at[c]],
                                      rowsb.at[pl.ds(0, CK)], sem0).wait()

            @pl.when(jnp.logical_not(even))
            def _():
                pltpu.make_async_copy(mem_hbm.at[idxv.at[c]],
                                      rowsb.at[pl.ds(CK, CK)], sem1).wait()

            h = par * CK
            for g in range(CK // 16):
                r0 = g * 16
                for r in range(16):
                    acc = _tree_sum(
                        rowsb[h + r0 + r, pl.ds(16 * j, 16)] * xs[j]
                        for j in range(NJ))
                    plsc.store_scatter(part, [lane + 17 * r], acc)
                dots = _tree_sum(
                    plsc.load_gather(part, [lane * 17 + cc])
                    for cc in range(16))
                stage[pl.ds(c * CK + r0, 16)] = dots
            return 0

        lax.fori_loop(0, NCHUNK, chunk_body, 0)
        pltpu.sync_copy(stage, out_hbm.at[row])
        return 0

    lax.fori_loop(0, BPW, per_b, 0)


_sc_dots = functools.partial(
    pl.kernel,
    out_type=jax.ShapeDtypeStruct((BATCH, K_PAD), jnp.float32),
    mesh=plsc.VectorSubcoreMesh(
        core_axis_name="c", subcore_axis_name="s", num_cores=NC,
        num_subcores=NS),
    scratch_types=[
        pltpu.VMEM((BPW, DIM), jnp.float32),
        pltpu.VMEM((NCHUNK, CK), jnp.int32),
        pltpu.VMEM((2 * CK, DIM), jnp.float32),
        pltpu.VMEM((17 * 16,), jnp.float32),
        pltpu.VMEM((K_PAD,), jnp.float32),
        pltpu.SemaphoreType.DMA,
        pltpu.SemaphoreType.DMA,
    ],
    compiler_params=pltpu.CompilerParams(needs_layout_passes=False),
)(_sc_body)


def _finish_body(x_ref, raw_ref, out_ref):
    x = x_ref[...]
    raw = raw_ref[...]
    nrm = jnp.sqrt(jnp.sum(x * x, axis=1, keepdims=True))
    inv = 1.0 / (jnp.maximum(nrm, 1e-12) * TEMP)
    s = raw * inv
    col = lax.broadcasted_iota(jnp.int32, s.shape, 1)
    valid = col < K
    sm = jnp.where(valid, s, -jnp.inf)
    m = jnp.max(sm, axis=1, keepdims=True)
    e = jnp.where(valid, jnp.exp(sm - m), 0.0)
    lse = jnp.log(jnp.sum(e, axis=1, keepdims=True)) + m
    out_ref[0, 0] = jnp.mean(lse - s[:, 0:1])


_finish = pl.pallas_call(
    _finish_body,
    out_shape=jax.ShapeDtypeStruct((1, 1), jnp.float32),
    out_specs=pl.BlockSpec(memory_space=pltpu.SMEM),
)


def kernel(x, x_ind, idx, memory):
    idx_full = jnp.concatenate([x_ind[:, None], idx[:, 1:]], axis=1)
    npad = K_PAD - K
    pad = jnp.arange(BATCH * npad, dtype=jnp.int32).reshape(BATCH, npad)
    idx_p = jnp.concatenate([idx_full, pad], axis=1)
    idx_p = idx_p.reshape(BATCH, NCHUNK, CK)
    raw = _sc_dots(x, idx_p, memory)
    return _finish(x, raw)[0, 0]


# speedup vs baseline: 1.6205x; 1.6205x over previous
import functools

import jax
import jax.numpy as jnp
from jax import lax
from jax.experimental import pallas as pl
from jax.experimental.pallas import tpu as pltpu
from jax.experimental.pallas import tpu_sc as plsc

BANK_SIZE = 500000
DIM = 256
NEG_SIZE = 2048
BATCH = 256
TEMP = 0.07

K = NEG_SIZE + 1
CK = 96
NCHUNK = 22
K_PAD = CK * NCHUNK
NC = 2
NS = 16
NW = NC * NS
BPW = BATCH // NW
NJ = DIM // 16


def _tree_sum(vals):
    vals = list(vals)
    while len(vals) > 1:
        nxt = [vals[i] + vals[i + 1] for i in range(0, len(vals) - 1, 2)]
        if len(vals) % 2:
            nxt.append(vals[-1])
        vals = nxt
    return vals[0]


def _sc_body(x_hbm, idx_hbm, mem_hbm, out_hbm,
             xv, idxv, rows0, rows1, part, stage, sem0, sem1):
    cid = lax.axis_index("c")
    sid = lax.axis_index("s")
    wid = sid * NC + cid
    b0 = wid * BPW
    lane = lax.iota(jnp.int32, 16)
    npair = NCHUNK // 2

    pltpu.sync_copy(x_hbm.at[pl.ds(b0, BPW)], xv)

    def per_b(lb, _):
        row = b0 + lb
        pltpu.sync_copy(idx_hbm.at[row], idxv)
        xs = [xv[lb, pl.ds(16 * j, 16)] for j in range(NJ)]

        def compute(rows, c):
            def group_body(g2, _):
                for half in range(2):
                    r0 = g2 * 32 + half * 16
                    pb = half * 272
                    for r in range(16):
                        acc = _tree_sum(
                            rows[r0 + r, pl.ds(16 * j, 16)] * xs[j]
                            for j in range(NJ))
                        plsc.store_scatter(part, [lane + (pb + 17 * r)], acc)
                    dots = _tree_sum(
                        plsc.load_gather(part, [lane * 17 + (pb + cc)])
                        for cc in range(16))
                    stage[pl.ds(c * CK + r0, 16)] = dots
                return 0

            lax.fori_loop(0, CK // 32, group_body, 0)

        pltpu.async_copy(mem_hbm.at[idxv.at[0]], rows0, sem0)

        def pair_body(p, _):
            c0 = 2 * p
            pltpu.async_copy(mem_hbm.at[idxv.at[c0 + 1]], rows1, sem1)
            pltpu.make_async_copy(mem_hbm.at[idxv.at[c0]], rows0, sem0).wait()
            compute(rows0, c0)

            @pl.when(p < npair - 1)
            def _():
                pltpu.async_copy(mem_hbm.at[idxv.at[c0 + 2]], rows0, sem0)

            pltpu.make_async_copy(
                mem_hbm.at[idxv.at[c0 + 1]], rows1, sem1).wait()
            compute(rows1, c0 + 1)
            return 0

        lax.fori_loop(0, npair, pair_body, 0)
        pltpu.sync_copy(stage, out_hbm.at[row])
        return 0

    lax.fori_loop(0, BPW, per_b, 0)


_sc_dots = functools.partial(
    pl.kernel,
    out_type=jax.ShapeDtypeStruct((BATCH, K_PAD), jnp.float32),
    mesh=plsc.VectorSubcoreMesh(
        core_axis_name="c", subcore_axis_name="s", num_cores=NC,
        num_subcores=NS),
    scratch_types=[
        pltpu.VMEM((BPW, DIM), jnp.float32),
        pltpu.VMEM((NCHUNK, CK), jnp.int32),
        pltpu.VMEM((CK, DIM), jnp.float32),
        pltpu.VMEM((CK, DIM), jnp.float32),
        pltpu.VMEM((2 * 17 * 16,), jnp.float32),
        pltpu.VMEM((K_PAD,), jnp.float32),
        pltpu.SemaphoreType.DMA,
        pltpu.SemaphoreType.DMA,
    ],
    compiler_params=pltpu.CompilerParams(needs_layout_passes=False),
)(_sc_body)


def _finish_body(x_ref, raw_ref, out_ref):
    x = x_ref[...]
    raw = raw_ref[...]
    nrm = jnp.sqrt(jnp.sum(x * x, axis=1, keepdims=True))
    inv = 1.0 / (jnp.maximum(nrm, 1e-12) * TEMP)
    s = raw * inv
    col = lax.broadcasted_iota(jnp.int32, s.shape, 1)
    valid = col < K
    sm = jnp.where(valid, s, -jnp.inf)
    m = jnp.max(sm, axis=1, keepdims=True)
    e = jnp.where(valid, jnp.exp(sm - m), 0.0)
    lse = jnp.log(jnp.sum(e, axis=1, keepdims=True)) + m
    out_ref[0, 0] = jnp.mean(lse - s[:, 0:1])


_finish = pl.pallas_call(
    _finish_body,
    out_shape=jax.ShapeDtypeStruct((1, 1), jnp.float32),
    out_specs=pl.BlockSpec(memory_space=pltpu.SMEM),
)


def kernel(x, x_ind, idx, memory):
    idx_full = jnp.concatenate([x_ind[:, None], idx[:, 1:]], axis=1)
    npad = K_PAD - K
    pad = jnp.arange(BATCH * npad, dtype=jnp.int32).reshape(BATCH, npad)
    idx_p = jnp.concatenate([idx_full, pad], axis=1)
    idx_p = idx_p.reshape(BATCH, NCHUNK, CK)
    raw = _sc_dots(x, idx_p, memory)
    return _finish(x, raw)[0, 0]


# speedup vs baseline: 3.0841x; 1.9032x over previous
import functools

import jax
import jax.numpy as jnp
from jax import lax
from jax.experimental import pallas as pl
from jax.experimental.pallas import tpu as pltpu
from jax.experimental.pallas import tpu_sc as plsc

BANK_SIZE = 500000
DIM = 256
NEG_SIZE = 2048
BATCH = 256
TEMP = 0.07

K = NEG_SIZE + 1
CK = 96
NCHUNK = 22
K_PAD = CK * NCHUNK
NC = 2
NS = 16
NW = NC * NS
BPW = BATCH // NW
NJ = DIM // 16


def _tree_sum(vals):
    vals = list(vals)
    while len(vals) > 1:
        nxt = [vals[i] + vals[i + 1] for i in range(0, len(vals) - 1, 2)]
        if len(vals) % 2:
            nxt.append(vals[-1])
        vals = nxt
    return vals[0]


def _sc_body(x_hbm, idx_hbm, mem_hbm, out_hbm,
             xv, idxv, rows0, rows1, part, stage, sem0, sem1):
    cid = lax.axis_index("c")
    sid = lax.axis_index("s")
    wid = sid * NC + cid
    b0 = wid * BPW
    lane = lax.iota(jnp.int32, 16)
    npair = NCHUNK // 2

    pltpu.sync_copy(x_hbm.at[pl.ds(b0, BPW)], xv)

    def per_b(lb, _):
        row = b0 + lb
        pltpu.sync_copy(idx_hbm.at[row], idxv)
        xs = [xv[lb, pl.ds(16 * j, 16)] for j in range(NJ)]

        def compute(rows, c):
            def group_body(g, _):
                r0 = g * 16

                @plsc.parallel_loop(0, 16, unroll=2)
                def row_body(r):
                    acc = _tree_sum(
                        rows[r0 + r, pl.ds(16 * j, 16)] * xs[j]
                        for j in range(NJ))
                    plsc.store_scatter(part, [lane + 17 * r], acc)

                dots = _tree_sum(
                    plsc.load_gather(part, [lane * 17 + cc])
                    for cc in range(16))
                stage[pl.ds(c * CK + r0, 16)] = dots
                return 0

            lax.fori_loop(0, CK // 16, group_body, 0)

        pltpu.async_copy(mem_hbm.at[idxv.at[0]], rows0, sem0)

        def pair_body(p, _):
            c0 = 2 * p
            pltpu.async_copy(mem_hbm.at[idxv.at[c0 + 1]], rows1, sem1)
            pltpu.make_async_copy(mem_hbm.at[idxv.at[c0]], rows0, sem0).wait()
            compute(rows0, c0)

            @pl.when(p < npair - 1)
            def _():
                pltpu.async_copy(mem_hbm.at[idxv.at[c0 + 2]], rows0, sem0)

            pltpu.make_async_copy(
                mem_hbm.at[idxv.at[c0 + 1]], rows1, sem1).wait()
            compute(rows1, c0 + 1)
            return 0

        lax.fori_loop(0, npair, pair_body, 0)
        pltpu.sync_copy(stage, out_hbm.at[row])
        return 0

    lax.fori_loop(0, BPW, per_b, 0)


_sc_dots = functools.partial(
    pl.kernel,
    out_type=jax.ShapeDtypeStruct((BATCH, K_PAD), jnp.float32),
    mesh=plsc.VectorSubcoreMesh(
        core_axis_name="c", subcore_axis_name="s", num_cores=NC,
        num_subcores=NS),
    scratch_types=[
        pltpu.VMEM((BPW, DIM), jnp.float32),
        pltpu.VMEM((NCHUNK, CK), jnp.int32),
        pltpu.VMEM((CK, DIM), jnp.float32),
        pltpu.VMEM((CK, DIM), jnp.float32),
        pltpu.VMEM((2 * 17 * 16,), jnp.float32),
        pltpu.VMEM((K_PAD,), jnp.float32),
        pltpu.SemaphoreType.DMA,
        pltpu.SemaphoreType.DMA,
    ],
    compiler_params=pltpu.CompilerParams(needs_layout_passes=False),
)(_sc_body)


def _finish_body(x_ref, raw_ref, out_ref):
    x = x_ref[...]
    raw = raw_ref[...]
    nrm = jnp.sqrt(jnp.sum(x * x, axis=1, keepdims=True))
    inv = 1.0 / (jnp.maximum(nrm, 1e-12) * TEMP)
    s = raw * inv
    col = lax.broadcasted_iota(jnp.int32, s.shape, 1)
    valid = col < K
    sm = jnp.where(valid, s, -jnp.inf)
    m = jnp.max(sm, axis=1, keepdims=True)
    e = jnp.where(valid, jnp.exp(sm - m), 0.0)
    lse = jnp.log(jnp.sum(e, axis=1, keepdims=True)) + m
    out_ref[0, 0] = jnp.mean(lse - s[:, 0:1])


_finish = pl.pallas_call(
    _finish_body,
    out_shape=jax.ShapeDtypeStruct((1, 1), jnp.float32),
    out_specs=pl.BlockSpec(memory_space=pltpu.SMEM),
)


def kernel(x, x_ind, idx, memory):
    idx_full = jnp.concatenate([x_ind[:, None], idx[:, 1:]], axis=1)
    npad = K_PAD - K
    pad = jnp.arange(BATCH * npad, dtype=jnp.int32).reshape(BATCH, npad)
    idx_p = jnp.concatenate([idx_full, pad], axis=1)
    idx_p = idx_p.reshape(BATCH, NCHUNK, CK)
    raw = _sc_dots(x, idx_p, memory)
    return _finish(x, raw)[0, 0]


# speedup vs baseline: 3.0894x; 1.0017x over previous
import functools

import jax
import jax.numpy as jnp
from jax import lax
from jax.experimental import pallas as pl
from jax.experimental.pallas import tpu as pltpu
from jax.experimental.pallas import tpu_sc as plsc

BANK_SIZE = 500000
DIM = 256
NEG_SIZE = 2048
BATCH = 256
TEMP = 0.07

K = NEG_SIZE + 1
CK = 96
NCHUNK = 22
K_PAD = CK * NCHUNK
NC = 2
NS = 16
NW = NC * NS
BPW = BATCH // NW
NJ = DIM // 16


def _tree_sum(vals):
    vals = list(vals)
    while len(vals) > 1:
        nxt = [vals[i] + vals[i + 1] for i in range(0, len(vals) - 1, 2)]
        if len(vals) % 2:
            nxt.append(vals[-1])
        vals = nxt
    return vals[0]


def _sc_body(x_hbm, idx_hbm, mem_hbm, out_hbm,
             xv, idxv, rows0, rows1, part, stage, sem0, sem1):
    cid = lax.axis_index("c")
    sid = lax.axis_index("s")
    wid = sid * NC + cid
    b0 = wid * BPW
    lane = lax.iota(jnp.int32, 16)
    npair = NCHUNK // 2

    pltpu.sync_copy(x_hbm.at[pl.ds(b0, BPW)], xv)

    def per_b(lb, _):
        row = b0 + lb
        pltpu.sync_copy(idx_hbm.at[row], idxv)
        xs = [xv[lb, pl.ds(16 * j, 16)] for j in range(NJ)]

        def compute(rows, c):
            def group_body(g, _):
                r0 = g * 16

                @plsc.parallel_loop(0, 16, unroll=4)
                def row_body(r):
                    acc = _tree_sum(
                        rows[r0 + r, pl.ds(16 * j, 16)] * xs[j]
                        for j in range(NJ))
                    plsc.store_scatter(part, [lane + 17 * r], acc)

                dots = _tree_sum(
                    plsc.load_gather(part, [lane * 17 + cc])
                    for cc in range(16))
                stage[pl.ds(c * CK + r0, 16)] = dots
                return 0

            lax.fori_loop(0, CK // 16, group_body, 0)

        pltpu.async_copy(mem_hbm.at[idxv.at[0]], rows0, sem0)

        def pair_body(p, _):
            c0 = 2 * p
            pltpu.async_copy(mem_hbm.at[idxv.at[c0 + 1]], rows1, sem1)
            pltpu.make_async_copy(mem_hbm.at[idxv.at[c0]], rows0, sem0).wait()
            compute(rows0, c0)

            @pl.when(p < npair - 1)
            def _():
                pltpu.async_copy(mem_hbm.at[idxv.at[c0 + 2]], rows0, sem0)

            pltpu.make_async_copy(
                mem_hbm.at[idxv.at[c0 + 1]], rows1, sem1).wait()
            compute(rows1, c0 + 1)
            return 0

        lax.fori_loop(0, npair, pair_body, 0)
        pltpu.sync_copy(stage, out_hbm.at[row])
        return 0

    lax.fori_loop(0, BPW, per_b, 0)


_sc_dots = functools.partial(
    pl.kernel,
    out_type=jax.ShapeDtypeStruct((BATCH, K_PAD), jnp.float32),
    mesh=plsc.VectorSubcoreMesh(
        core_axis_name="c", subcore_axis_name="s", num_cores=NC,
        num_subcores=NS),
    scratch_types=[
        pltpu.VMEM((BPW, DIM), jnp.float32),
        pltpu.VMEM((NCHUNK, CK), jnp.int32),
        pltpu.VMEM((CK, DIM), jnp.float32),
        pltpu.VMEM((CK, DIM), jnp.float32),
        pltpu.VMEM((2 * 17 * 16,), jnp.float32),
        pltpu.VMEM((K_PAD,), jnp.float32),
        pltpu.SemaphoreType.DMA,
        pltpu.SemaphoreType.DMA,
    ],
    compiler_params=pltpu.CompilerParams(needs_layout_passes=False),
)(_sc_body)


def _finish_body(x_ref, raw_ref, out_ref):
    x = x_ref[...]
    raw = raw_ref[...]
    nrm = jnp.sqrt(jnp.sum(x * x, axis=1, keepdims=True))
    inv = 1.0 / (jnp.maximum(nrm, 1e-12) * TEMP)
    s = raw * inv
    col = lax.broadcasted_iota(jnp.int32, s.shape, 1)
    valid = col < K
    sm = jnp.where(valid, s, -jnp.inf)
    m = jnp.max(sm, axis=1, keepdims=True)
    e = jnp.where(valid, jnp.exp(sm - m), 0.0)
    lse = jnp.log(jnp.sum(e, axis=1, keepdims=True)) + m
    out_ref[0, 0] = jnp.mean(lse - s[:, 0:1])


_finish = pl.pallas_call(
    _finish_body,
    out_shape=jax.ShapeDtypeStruct((1, 1), jnp.float32),
    out_specs=pl.BlockSpec(memory_space=pltpu.SMEM),
)


def kernel(x, x_ind, idx, memory):
    idx_full = jnp.concatenate([x_ind[:, None], idx[:, 1:]], axis=1)
    npad = K_PAD - K
    pad = jnp.arange(BATCH * npad, dtype=jnp.int32).reshape(BATCH, npad)
    idx_p = jnp.concatenate([idx_full, pad], axis=1)
    idx_p = idx_p.reshape(BATCH, NCHUNK, CK)
    raw = _sc_dots(x, idx_p, memory)
    return _finish(x, raw)[0, 0]


# speedup vs baseline: 3.3077x; 1.0707x over previous
import functools

import jax
import jax.numpy as jnp
from jax import lax
from jax.experimental import pallas as pl
from jax.experimental.pallas import tpu as pltpu
from jax.experimental.pallas import tpu_sc as plsc

BANK_SIZE = 500000
DIM = 256
NEG_SIZE = 2048
BATCH = 256
TEMP = 0.07

K = NEG_SIZE + 1
CK = 96
NCHUNK = 22
K_PAD = CK * NCHUNK
NC = 2
NS = 16
NW = NC * NS
BPW = BATCH // NW
NJ = DIM // 16


def _tree_sum(vals):
    vals = list(vals)
    while len(vals) > 1:
        nxt = [vals[i] + vals[i + 1] for i in range(0, len(vals) - 1, 2)]
        if len(vals) % 2:
            nxt.append(vals[-1])
        vals = nxt
    return vals[0]


def _sc_body(x_hbm, idx_hbm, mem_hbm, out_hbm,
             xv, idxv, rows0, rows1, part, stage, sem0, sem1, semo0, semo1):
    cid = lax.axis_index("c")
    sid = lax.axis_index("s")
    wid = sid * NC + cid
    b0 = wid * BPW
    lane = lax.iota(jnp.int32, 16)
    npair = NCHUNK // 2

    pltpu.sync_copy(x_hbm.at[pl.ds(b0, BPW)], xv)
    pltpu.sync_copy(idx_hbm.at[pl.ds(b0, BPW)], idxv)

    pltpu.async_copy(mem_hbm.at[idxv.at[0, 0]], rows0, sem0)

    def per_b(lb, _):
        row = b0 + lb
        par = lax.rem(lb, 2)
        xs = [xv[lb, pl.ds(16 * j, 16)] for j in range(NJ)]

        @pl.when((lb >= 2) & (par == 0))
        def _():
            pltpu.make_async_copy(
                stage.at[0], out_hbm.at[row], semo0).wait()

        @pl.when((lb >= 2) & (par == 1))
        def _():
            pltpu.make_async_copy(
                stage.at[1], out_hbm.at[row], semo1).wait()

        def compute(rows, c):
            def group_body(g, _):
                r0 = g * 16

                @plsc.parallel_loop(0, 16, unroll=4)
                def row_body(r):
                    acc = _tree_sum(
                        rows[r0 + r, pl.ds(16 * j, 16)] * xs[j]
                        for j in range(NJ))
                    plsc.store_scatter(part, [lane + 17 * r], acc)

                dots = _tree_sum(
                    plsc.load_gather(part, [lane * 17 + cc])
                    for cc in range(16))
                stage[par, pl.ds(c * CK + r0, 16)] = dots
                return 0

            lax.fori_loop(0, CK // 16, group_body, 0)

        def pair_body(p, _):
            c0 = 2 * p
            pltpu.async_copy(mem_hbm.at[idxv.at[lb, c0 + 1]], rows1, sem1)
            pltpu.make_async_copy(
                mem_hbm.at[idxv.at[lb, c0]], rows0, sem0).wait()
            compute(rows0, c0)

            @pl.when(p < npair - 1)
            def _():
                pltpu.async_copy(mem_hbm.at[idxv.at[lb, c0 + 2]], rows0,
                                 sem0)

            @pl.when((p == npair - 1) & (lb < BPW - 1))
            def _():
                pltpu.async_copy(mem_hbm.at[idxv.at[lb + 1, 0]], rows0,
                                 sem0)

            pltpu.make_async_copy(
                mem_hbm.at[idxv.at[lb, c0 + 1]], rows1, sem1).wait()
            compute(rows1, c0 + 1)
            return 0

        lax.fori_loop(0, npair, pair_body, 0)

        @pl.when(par == 0)
        def _():
            pltpu.async_copy(
                stage.at[0], out_hbm.at[row], semo0)

        @pl.when(par == 1)
        def _():
            pltpu.async_copy(
                stage.at[1], out_hbm.at[row], semo1)

        return 0

    lax.fori_loop(0, BPW, per_b, 0)

    pltpu.make_async_copy(
        stage.at[0], out_hbm.at[b0 + BPW - 2], semo0).wait()
    pltpu.make_async_copy(
        stage.at[1], out_hbm.at[b0 + BPW - 1],
        semo1).wait()


_sc_dots = functools.partial(
    pl.kernel,
    out_type=jax.ShapeDtypeStruct((BATCH, K_PAD), jnp.float32),
    mesh=plsc.VectorSubcoreMesh(
        core_axis_name="c", subcore_axis_name="s", num_cores=NC,
        num_subcores=NS),
    scratch_types=[
        pltpu.VMEM((BPW, DIM), jnp.float32),
        pltpu.VMEM((BPW, NCHUNK, CK), jnp.int32),
        pltpu.VMEM((CK, DIM), jnp.float32),
        pltpu.VMEM((CK, DIM), jnp.float32),
        pltpu.VMEM((2 * 17 * 16,), jnp.float32),
        pltpu.VMEM((2, K_PAD), jnp.float32),
        pltpu.SemaphoreType.DMA,
        pltpu.SemaphoreType.DMA,
        pltpu.SemaphoreType.DMA,
        pltpu.SemaphoreType.DMA,
    ],
    compiler_params=pltpu.CompilerParams(needs_layout_passes=False),
)(_sc_body)


def _finish_body(x_ref, raw_ref, out_ref):
    x = x_ref[...]
    raw = raw_ref[...]
    nrm = jnp.sqrt(jnp.sum(x * x, axis=1, keepdims=True))
    inv = 1.0 / (jnp.maximum(nrm, 1e-12) * TEMP)
    s = raw * inv
    col = lax.broadcasted_iota(jnp.int32, s.shape, 1)
    valid = col < K
    sm = jnp.where(valid, s, -jnp.inf)
    m = jnp.max(sm, axis=1, keepdims=True)
    e = jnp.where(valid, jnp.exp(sm - m), 0.0)
    lse = jnp.log(jnp.sum(e, axis=1, keepdims=True)) + m
    out_ref[0, 0] = jnp.mean(lse - s[:, 0:1])


_finish = pl.pallas_call(
    _finish_body,
    out_shape=jax.ShapeDtypeStruct((1, 1), jnp.float32),
    out_specs=pl.BlockSpec(memory_space=pltpu.SMEM),
)


def kernel(x, x_ind, idx, memory):
    idx_full = jnp.concatenate([x_ind[:, None], idx[:, 1:]], axis=1)
    npad = K_PAD - K
    pad = jnp.arange(BATCH * npad, dtype=jnp.int32).reshape(BATCH, npad)
    idx_p = jnp.concatenate([idx_full, pad], axis=1)
    idx_p = idx_p.reshape(BATCH, NCHUNK, CK)
    raw = _sc_dots(x, idx_p, memory)
    return _finish(x, raw)[0, 0]
